# baseline (speedup 1.0000x reference)
import functools

import jax
import jax.numpy as jnp
from jax import lax
from jax.experimental import pallas as pl
from jax.experimental.pallas import tpu as pltpu
from jax.experimental.pallas import tpu_sc as plsc


def _sc_gather(item_table, iidx, user_table, uidx):
    BL = iidx.shape[0]
    B = uidx.shape[0]
    D = item_table.shape[1]

    info = plsc.get_sparse_core_info()
    nw = info.num_cores * info.num_subcores
    nc = info.num_cores
    ipw = BL // nw
    upw = B // nw
    assert BL % nw == 0 and B % nw == 0 and ipw % 8 == 0 and upw % 8 == 0
    CH = 64
    n_ch = ipw // CH
    assert ipw % CH == 0

    mesh = plsc.VectorSubcoreMesh(core_axis_name="c", subcore_axis_name="s")

    @functools.partial(
        pl.kernel,
        mesh=mesh,
        out_type=[
            jax.ShapeDtypeStruct((BL, D), jnp.float32),
            jax.ShapeDtypeStruct((B, D), jnp.float32),
        ],
        scratch_types=[
            pltpu.VMEM((ipw,), jnp.int32),
            pltpu.VMEM((ipw, D), jnp.float32),
            pltpu.VMEM((upw,), jnp.int32),
            pltpu.VMEM((upw, D), jnp.float32),
            pltpu.SemaphoreType.DMA,
        ],
        compiler_params=pltpu.CompilerParams(use_tc_tiling_on_sc=False),
    )
    def gather_k(item_hbm, iidx_hbm, user_hbm, uidx_hbm, hist_out, urep_out,
                 iidx_v, rows_v, uidx_v, urows_v, sem):
        wid = lax.axis_index("s") * nc + lax.axis_index("c")
        base = wid * ipw
        ubase = wid * upw

        pltpu.sync_copy(iidx_hbm.at[pl.ds(base, ipw)], iidx_v)
        pltpu.sync_copy(uidx_hbm.at[pl.ds(ubase, upw)], uidx_v)

        pltpu.make_async_copy(user_hbm.at[uidx_v], urows_v, sem).start()

        def fire(c, carry):
            pltpu.make_async_copy(
                item_hbm.at[iidx_v.at[pl.ds(c * CH, CH)]],
                rows_v.at[pl.ds(c * CH, CH)],
                sem,
            ).start()
            return carry

        lax.fori_loop(0, n_ch, fire, 0)

        pltpu.make_async_copy(user_hbm.at[uidx_v], urows_v, sem).wait()
        pltpu.sync_copy(urows_v, urep_out.at[pl.ds(ubase, upw)])

        def drain(c, carry):
            pltpu.make_async_copy(
                item_hbm.at[iidx_v.at[pl.ds(0, CH)]],
                rows_v.at[pl.ds(0, CH)],
                sem,
            ).wait()
            return carry

        lax.fori_loop(0, n_ch, drain, 0)
        pltpu.sync_copy(rows_v, hist_out.at[pl.ds(base, ipw)])

    return gather_k(item_table, iidx, user_table, uidx)


def _dot_t(x, w):
    return lax.dot_general(x, w, (((1,), (1,)), ((), ())),
                           preferred_element_type=jnp.float32)


def _dot(x, w):
    return lax.dot_general(x, w, (((1,), (0,)), ((), ())),
                           preferred_element_type=jnp.float32)


def _tc_body(L, D,
             hist_ref, rat_ref, urep_ref, op_ref, w1_ref, b1_ref, w2_ref,
             b2_ref, wa1_ref, ba1_ref, wa2_ref, ba2_ref, wa3_ref, ba3_ref,
             out_ref):
    hist = hist_ref[...]
    R = hist.shape[0]
    BB = R // L

    w1 = w1_ref[...]
    r1 = _dot_t(op_ref[...], w1[:, D:])

    x1 = _dot_t(hist, w1[:, :D])
    rat = rat_ref[...]
    r_c = jnp.zeros_like(x1)
    for r in range(r1.shape[0]):
        m = (rat == r).astype(jnp.float32)
        r_c = r_c + m * r1[r:r + 1, :]
    x1 = jnp.maximum(x1 + r_c + b1_ref[...], 0.0)

    o = jnp.maximum(_dot_t(x1, w2_ref[...]) + b2_ref[...], 0.0)

    wa1 = wa1_ref[...]
    u_c = _dot_t(urep_ref[...], wa1[:, D:])
    rows = lax.broadcasted_iota(jnp.int32, (R, BB), 0) // L
    cols = lax.broadcasted_iota(jnp.int32, (R, BB), 1)
    rep = (rows == cols).astype(jnp.float32)
    a1 = jnp.maximum(_dot_t(o, wa1[:, :D]) + _dot(rep, u_c) + ba1_ref[...], 0.0)
    a2 = jnp.maximum(_dot_t(a1, wa2_ref[...]) + ba2_ref[...], 0.0)
    s = jnp.sum(a2 * wa3_ref[...], axis=1, keepdims=True) + ba3_ref[...]

    e = jnp.exp(s - jnp.max(s))
    srows = lax.broadcasted_iota(jnp.int32, (BB, R), 0)
    scols = lax.broadcasted_iota(jnp.int32, (BB, R), 1) // L
    seg = (srows == scols).astype(jnp.float32)
    num = _dot(seg, o * e)
    den = _dot(seg, e)
    out_ref[...] = num / den


def _tc_forward(hist, rating_rows, urep, opinion,
                W1, b1, W2, b2, Wa1, ba1, Wa2, ba2, Wa3, ba3, BB=128):
    B, D = urep.shape
    L = hist.shape[0] // B
    grid = (B // BB,)
    R = BB * L

    def full(shape):
        return pl.BlockSpec(shape, lambda i: (0, 0))

    return pl.pallas_call(
        functools.partial(_tc_body, L, D),
        grid=grid,
        in_specs=[
            pl.BlockSpec((R, D), lambda i: (i, 0)),
            pl.BlockSpec((R, 1), lambda i: (i, 0)),
            pl.BlockSpec((BB, D), lambda i: (i, 0)),
            full(opinion.shape),
            full(W1.shape), full(b1.shape),
            full(W2.shape), full(b2.shape),
            full(Wa1.shape), full(ba1.shape),
            full(Wa2.shape), full(ba2.shape),
            full(Wa3.shape), full(ba3.shape),
        ],
        out_specs=pl.BlockSpec((BB, D), lambda i: (i, 0)),
        out_shape=jax.ShapeDtypeStruct((B, D), jnp.float32),
        compiler_params=pltpu.CompilerParams(
            dimension_semantics=("arbitrary",),
        ),
    )(hist, rating_rows, urep, opinion,
      W1, b1, W2, b2, Wa1, ba1, Wa2, ba2, Wa3, ba3)


def kernel(nodes, user_item_pair, rating, item_table, user_table,
           opinion_table, W1, b1, W2, b2, Wa1, ba1, Wa2, ba2, Wa3, ba3):
    B, L = user_item_pair.shape
    D = item_table.shape[1]

    iidx = user_item_pair.reshape(B * L).astype(jnp.int32)
    uidx = nodes.astype(jnp.int32)
    hist, urep = _sc_gather(item_table, iidx, user_table, uidx)

    rating_rows = rating.reshape(B * L, 1).astype(jnp.int32)
    return _tc_forward(
        hist, rating_rows, urep, opinion_table,
        W1, b1.reshape(1, D), W2, b2.reshape(1, D),
        Wa1, ba1.reshape(1, D), Wa2, ba2.reshape(1, D),
        Wa3, ba3.reshape(1, 1),
    )


# speedup vs baseline: 1.2485x; 1.2485x over previous
import functools

import jax
import jax.numpy as jnp
from jax import lax
from jax.experimental import pallas as pl
from jax.experimental.pallas import tpu as pltpu
from jax.experimental.pallas import tpu_sc as plsc


def _tc_transform(table_t, w1h, nb=4096):
    D, V = table_t.shape
    grid = (pl.cdiv(V, nb),)

    def body(t_ref, w_ref, out_ref):
        out_ref[...] = lax.dot_general(
            t_ref[...], w_ref[...], (((0,), (1,)), ((), ())),
            preferred_element_type=jnp.float32)

    return pl.pallas_call(
        body,
        grid=grid,
        in_specs=[
            pl.BlockSpec((D, nb), lambda i: (0, i)),
            pl.BlockSpec((D, D), lambda i: (0, 0)),
        ],
        out_specs=pl.BlockSpec((nb, D), lambda i: (i, 0)),
        out_shape=jax.ShapeDtypeStruct((V, D), jnp.float32),
        compiler_params=pltpu.CompilerParams(
            dimension_semantics=("arbitrary",),
        ),
    )(table_t, w1h)


def _sc_row_gather(g, iidx):
    V, D = g.shape
    BL = iidx.shape[0]

    info = plsc.get_sparse_core_info()
    nw = info.num_cores * info.num_subcores
    nc = info.num_cores
    ipw = BL // nw
    assert BL % nw == 0 and ipw % 8 == 0
    CH = 64
    n_ch = ipw // CH
    assert ipw % CH == 0

    mesh = plsc.VectorSubcoreMesh(core_axis_name="c", subcore_axis_name="s")

    @functools.partial(
        pl.kernel,
        mesh=mesh,
        out_type=jax.ShapeDtypeStruct((BL, D), jnp.float32),
        scratch_types=[
            pltpu.VMEM((ipw,), jnp.int32),
            pltpu.VMEM((ipw, D), jnp.float32),
            pltpu.SemaphoreType.DMA,
        ],
        compiler_params=pltpu.CompilerParams(use_tc_tiling_on_sc=False),
    )
    def gather_k(g_hbm, iidx_hbm, hist_out, iidx_v, rows_v, sem):
        wid = lax.axis_index("s") * nc + lax.axis_index("c")
        base = wid * ipw

        pltpu.sync_copy(iidx_hbm.at[pl.ds(base, ipw)], iidx_v)

        def fire(c, carry):
            pltpu.make_async_copy(
                g_hbm.at[iidx_v.at[pl.ds(c * CH, CH)]],
                rows_v.at[pl.ds(c * CH, CH)],
                sem,
            ).start()
            return carry

        lax.fori_loop(0, n_ch, fire, 0)

        def drain(c, carry):
            pltpu.make_async_copy(
                g_hbm.at[iidx_v.at[pl.ds(0, CH)]],
                rows_v.at[pl.ds(0, CH)],
                sem,
            ).wait()
            return carry

        lax.fori_loop(0, n_ch, drain, 0)
        pltpu.sync_copy(rows_v, hist_out.at[pl.ds(base, ipw)])

    return gather_k(g, iidx)


def _sc_user_gather(user_t, uidx):
    D, V = user_t.shape
    B = uidx.shape[0]
    CH = 128
    n_ch = B // CH
    assert B % CH == 0

    info = plsc.get_sparse_core_info()
    nw = info.num_cores * info.num_subcores
    nc = info.num_cores

    mesh = plsc.VectorSubcoreMesh(core_axis_name="c", subcore_axis_name="s")

    @functools.partial(
        pl.kernel,
        mesh=mesh,
        out_type=jax.ShapeDtypeStruct((B * D,), jnp.float32),
        scratch_types=[
            pltpu.VMEM((CH,), jnp.int32),
            pltpu.VMEM((D, 128), jnp.float32),
            pltpu.VMEM((CH * D,), jnp.float32),
            pltpu.SemaphoreType.DMA,
        ],
        compiler_params=pltpu.CompilerParams(needs_layout_passes=False),
    )
    def user_k(user_hbm, uidx_hbm, out_hbm, idx_v, blk_v, stage_v, sem):
        wid = lax.axis_index("s") * nc + lax.axis_index("c")

        @pl.when(wid < n_ch)
        def _():
            pltpu.sync_copy(uidx_hbm.at[pl.ds(wid * CH, CH)], idx_v)

            def one_user(j, carry):
                vec = idx_v[pl.ds((j // 16) * 16, 16)]
                lane = lax.iota(jnp.int32, 16)
                r = jnp.max(jnp.where(lane == j % 16, vec, 0))
                off = pl.multiple_of((r // 128) * 128, 128)
                rl = r - off
                cp = pltpu.make_async_copy(
                    user_hbm.at[:, pl.ds(off, 128)], blk_v, sem,
                )
                cp.start()
                cp.wait()
                for k in range(D // 16):
                    rows = lax.iota(jnp.int32, 16) + (16 * k)
                    cols = jnp.full((16,), rl, jnp.int32)
                    vals = plsc.load_gather(blk_v, [rows, cols])
                    pos = lax.iota(jnp.int32, 16) + (j * D + 16 * k)
                    plsc.store_scatter(stage_v, [pos], vals)
                return carry

            lax.fori_loop(0, CH, one_user, 0)
            pltpu.sync_copy(stage_v, out_hbm.at[pl.ds(wid * CH * D, CH * D)])

    return user_k(user_t, uidx).reshape(B, D)


def _dot_t(x, w):
    return lax.dot_general(x, w, (((1,), (1,)), ((), ())),
                           preferred_element_type=jnp.float32)


def _dot(x, w):
    return lax.dot_general(x, w, (((1,), (0,)), ((), ())),
                           preferred_element_type=jnp.float32)


def _tc_body(L, D,
             x1p_ref, rat_ref, urep_ref, op_ref, w1_ref, b1_ref, w2_ref,
             b2_ref, wa1_ref, ba1_ref, wa2_ref, ba2_ref, wa3_ref, ba3_ref,
             out_ref):
    x1p = x1p_ref[...]
    R = x1p.shape[0]
    BB = R // L

    w1 = w1_ref[...]
    r1 = _dot_t(op_ref[...], w1[:, D:])

    rat = rat_ref[...]
    r_c = jnp.zeros_like(x1p)
    for r in range(r1.shape[0]):
        m = (rat == r).astype(jnp.float32)
        r_c = r_c + m * r1[r:r + 1, :]
    x1 = jnp.maximum(x1p + r_c + b1_ref[...], 0.0)

    o = jnp.maximum(_dot_t(x1, w2_ref[...]) + b2_ref[...], 0.0)

    wa1 = wa1_ref[...]
    u_c = _dot_t(urep_ref[...], wa1[:, D:])
    rows = lax.broadcasted_iota(jnp.int32, (R, BB), 0) // L
    cols = lax.broadcasted_iota(jnp.int32, (R, BB), 1)
    rep = (rows == cols).astype(jnp.float32)
    a1 = jnp.maximum(_dot_t(o, wa1[:, :D]) + _dot(rep, u_c) + ba1_ref[...], 0.0)
    a2 = jnp.maximum(_dot_t(a1, wa2_ref[...]) + ba2_ref[...], 0.0)
    s = jnp.sum(a2 * wa3_ref[...], axis=1, keepdims=True) + ba3_ref[...]

    e = jnp.exp(s - jnp.max(s))
    srows = lax.broadcasted_iota(jnp.int32, (BB, R), 0)
    scols = lax.broadcasted_iota(jnp.int32, (BB, R), 1) // L
    seg = (srows == scols).astype(jnp.float32)
    num = _dot(seg, o * e)
    den = _dot(seg, e)
    out_ref[...] = num / den


def _tc_forward(x1p, rating_rows, urep, opinion,
                W1, b1, W2, b2, Wa1, ba1, Wa2, ba2, Wa3, ba3, BB=128):
    B, D = urep.shape
    L = x1p.shape[0] // B
    grid = (B // BB,)
    R = BB * L

    def full(shape):
        return pl.BlockSpec(shape, lambda i: (0, 0))

    return pl.pallas_call(
        functools.partial(_tc_body, L, D),
        grid=grid,
        in_specs=[
            pl.BlockSpec((R, D), lambda i: (i, 0)),
            pl.BlockSpec((R, 1), lambda i: (i, 0)),
            pl.BlockSpec((BB, D), lambda i: (i, 0)),
            full(opinion.shape),
            full(W1.shape), full(b1.shape),
            full(W2.shape), full(b2.shape),
            full(Wa1.shape), full(ba1.shape),
            full(Wa2.shape), full(ba2.shape),
            full(Wa3.shape), full(ba3.shape),
        ],
        out_specs=pl.BlockSpec((BB, D), lambda i: (i, 0)),
        out_shape=jax.ShapeDtypeStruct((B, D), jnp.float32),
        compiler_params=pltpu.CompilerParams(
            dimension_semantics=("arbitrary",),
        ),
    )(x1p, rating_rows, urep, opinion,
      W1, b1, W2, b2, Wa1, ba1, Wa2, ba2, Wa3, ba3)


def kernel(nodes, user_item_pair, rating, item_table, user_table,
           opinion_table, W1, b1, W2, b2, Wa1, ba1, Wa2, ba2, Wa3, ba3):
    B, L = user_item_pair.shape
    D = item_table.shape[1]

    uidx = nodes.astype(jnp.int32)
    urep = _sc_user_gather(user_table.T, uidx)

    g = _tc_transform(item_table.T, W1[:, :D])
    iidx = user_item_pair.reshape(B * L).astype(jnp.int32)
    x1p = _sc_row_gather(g, iidx)

    rating_rows = rating.reshape(B * L, 1).astype(jnp.int32)
    return _tc_forward(
        x1p, rating_rows, urep, opinion_table,
        W1, b1.reshape(1, D), W2, b2.reshape(1, D),
        Wa1, ba1.reshape(1, D), Wa2, ba2.reshape(1, D),
        Wa3, ba3.reshape(1, 1),
    )


# speedup vs baseline: 3.0442x; 2.4383x over previous
import functools

import jax
import jax.numpy as jnp
from jax import lax
from jax.experimental import pallas as pl
from jax.experimental.pallas import tpu as pltpu
from jax.experimental.pallas import tpu_sc as plsc

_H = 524288


def _tc_transform(table_t, w1h, nb=8192):
    D, V = table_t.shape
    grid = (_H // nb,)

    n_hi_max = V // nb

    def body(lo_ref, hi_ref, w_ref, out_ref):
        w = w_ref[...]
        lo = lax.dot_general(lo_ref[...], w, (((0,), (1,)), ((), ())),
                             preferred_element_type=jnp.float32)
        hi = lax.dot_general(hi_ref[...], w, (((0,), (1,)), ((), ())),
                             preferred_element_type=jnp.float32)
        out_ref[...] = jnp.concatenate([lo, hi], axis=1)

    return pl.pallas_call(
        body,
        grid=grid,
        in_specs=[
            pl.BlockSpec((D, nb), lambda i: (0, i)),
            pl.BlockSpec((D, nb),
                         lambda i: (0, jnp.minimum(i + _H // nb, n_hi_max))),
            pl.BlockSpec((D, D), lambda i: (0, 0)),
        ],
        out_specs=pl.BlockSpec((nb, 2 * D), lambda i: (i, 0)),
        out_shape=jax.ShapeDtypeStruct((_H, 2 * D), jnp.float32),
        compiler_params=pltpu.CompilerParams(
            dimension_semantics=("parallel",),
        ),
    )(table_t, table_t, w1h)


def _sc_row_gather(g2, iidx):
    H, W = g2.shape
    BL = iidx.shape[0]

    info = plsc.get_sparse_core_info()
    nw = info.num_cores * info.num_subcores
    nc = info.num_cores
    ipw = BL // nw
    assert BL % nw == 0 and ipw % 16 == 0
    HALF = ipw // 2
    CH = 80
    n_ch = HALF // CH
    assert HALF % CH == 0

    mesh = plsc.VectorSubcoreMesh(core_axis_name="c", subcore_axis_name="s")

    @functools.partial(
        pl.kernel,
        mesh=mesh,
        out_type=jax.ShapeDtypeStruct((BL, W), jnp.float32),
        scratch_types=[
            pltpu.VMEM((ipw,), jnp.int32),
            pltpu.VMEM((HALF, W), jnp.float32),
            pltpu.SemaphoreType.DMA,
        ],
    )
    def gather_k(g_hbm, iidx_hbm, hist_out, iidx_v, rows_v, sem):
        wid = lax.axis_index("s") * nc + lax.axis_index("c")
        base = wid * ipw

        pltpu.sync_copy(iidx_hbm.at[pl.ds(base, ipw)], iidx_v)

        for h in range(2):
            def fire(c, carry):
                pltpu.make_async_copy(
                    g_hbm.at[iidx_v.at[pl.ds(h * HALF + c * CH, CH)]],
                    rows_v.at[pl.ds(c * CH, CH)],
                    sem,
                ).start()
                return carry

            lax.fori_loop(0, n_ch, fire, 0)

            def drain(c, carry):
                pltpu.make_async_copy(
                    g_hbm.at[iidx_v.at[pl.ds(0, CH)]],
                    rows_v.at[pl.ds(0, CH)],
                    sem,
                ).wait()
                return carry

            lax.fori_loop(0, n_ch, drain, 0)
            pltpu.sync_copy(rows_v, hist_out.at[pl.ds(base + h * HALF, HALF)])

    return gather_k(g2, iidx)


def _sc_user_gather(user_t, uidx):
    D, V = user_t.shape
    B = uidx.shape[0]

    info = plsc.get_sparse_core_info()
    nw = info.num_cores * info.num_subcores
    nc = info.num_cores
    CH = B // nw
    assert B % nw == 0

    mesh = plsc.VectorSubcoreMesh(core_axis_name="c", subcore_axis_name="s")

    @functools.partial(
        pl.kernel,
        mesh=mesh,
        out_type=jax.ShapeDtypeStruct((B * D,), jnp.float32),
        scratch_types=[
            pltpu.VMEM((CH,), jnp.int32),
            pltpu.VMEM((D, 128), jnp.float32),
            pltpu.VMEM((CH * D,), jnp.float32),
            pltpu.SemaphoreType.DMA,
        ],
        compiler_params=pltpu.CompilerParams(needs_layout_passes=False),
    )
    def user_k(user_hbm, uidx_hbm, out_hbm, idx_v, blk_v, stage_v, sem):
        wid = lax.axis_index("s") * nc + lax.axis_index("c")

        pltpu.sync_copy(uidx_hbm.at[pl.ds(wid * CH, CH)], idx_v)

        def one_user(j, carry):
            vec = idx_v[pl.ds((j // 16) * 16, 16)]
            lane = lax.iota(jnp.int32, 16)
            r = jnp.max(jnp.where(lane == j % 16, vec, 0))
            off = pl.multiple_of((r // 128) * 128, 128)
            rl = r - off
            cp = pltpu.make_async_copy(
                user_hbm.at[:, pl.ds(off, 128)], blk_v, sem,
            )
            cp.start()
            cp.wait()
            for k in range(D // 16):
                rows = lax.iota(jnp.int32, 16) + (16 * k)
                cols = jnp.full((16,), rl, jnp.int32)
                vals = plsc.load_gather(blk_v, [rows, cols])
                pos = lax.iota(jnp.int32, 16) + (j * D + 16 * k)
                plsc.store_scatter(stage_v, [pos], vals)
            return carry

        lax.fori_loop(0, CH, one_user, 0)
        pltpu.sync_copy(stage_v, out_hbm.at[pl.ds(wid * CH * D, CH * D)])

    return user_k(user_t, uidx).reshape(B, D)


def _dot_t(x, w):
    return lax.dot_general(x, w, (((1,), (1,)), ((), ())),
                           preferred_element_type=jnp.float32)


def _dot(x, w):
    return lax.dot_general(x, w, (((1,), (0,)), ((), ())),
                           preferred_element_type=jnp.float32)


def _tc_body(L, D,
             x1p_ref, half_ref, rat_ref, urep_ref, op_ref, w1_ref, b1_ref,
             w2_ref, b2_ref, wa1_ref, ba1_ref, wa2_ref, ba2_ref, wa3_ref,
             ba3_ref, out_ref):
    x1p2 = x1p_ref[...]
    R = x1p2.shape[0]
    BB = R // L

    x1p = jnp.where(half_ref[...] == 0, x1p2[:, :D], x1p2[:, D:])

    w1 = w1_ref[...]
    r1 = _dot_t(op_ref[...], w1[:, D:])

    rat = rat_ref[...]
    r_c = jnp.zeros_like(x1p)
    for r in range(r1.shape[0]):
        m = (rat == r).astype(jnp.float32)
        r_c = r_c + m * r1[r:r + 1, :]
    x1 = jnp.maximum(x1p + r_c + b1_ref[...], 0.0)

    o = jnp.maximum(_dot_t(x1, w2_ref[...]) + b2_ref[...], 0.0)

    wa1 = wa1_ref[...]
    u_c = _dot_t(urep_ref[...], wa1[:, D:])
    rows = lax.broadcasted_iota(jnp.int32, (R, BB), 0) // L
    cols = lax.broadcasted_iota(jnp.int32, (R, BB), 1)
    rep = (rows == cols).astype(jnp.float32)
    a1 = jnp.maximum(_dot_t(o, wa1[:, :D]) + _dot(rep, u_c) + ba1_ref[...], 0.0)
    a2 = jnp.maximum(_dot_t(a1, wa2_ref[...]) + ba2_ref[...], 0.0)
    s = jnp.sum(a2 * wa3_ref[...], axis=1, keepdims=True) + ba3_ref[...]

    e = jnp.exp(s - jnp.max(s))
    srows = lax.broadcasted_iota(jnp.int32, (BB, R), 0)
    scols = lax.broadcasted_iota(jnp.int32, (BB, R), 1) // L
    seg = (srows == scols).astype(jnp.float32)
    num = _dot(seg, o * e)
    den = _dot(seg, e)
    out_ref[...] = num / den


def _tc_forward(x1p2, half_rows, rating_rows, urep, opinion,
                W1, b1, W2, b2, Wa1, ba1, Wa2, ba2, Wa3, ba3, BB=128):
    B, D = urep.shape
    L = x1p2.shape[0] // B
    grid = (B // BB,)
    R = BB * L

    def full(shape):
        return pl.BlockSpec(shape, lambda i: (0, 0))

    return pl.pallas_call(
        functools.partial(_tc_body, L, D),
        grid=grid,
        in_specs=[
            pl.BlockSpec((R, 2 * D), lambda i: (i, 0)),
            pl.BlockSpec((R, 1), lambda i: (i, 0)),
            pl.BlockSpec((R, 1), lambda i: (i, 0)),
            pl.BlockSpec((BB, D), lambda i: (i, 0)),
            full(opinion.shape),
            full(W1.shape), full(b1.shape),
            full(W2.shape), full(b2.shape),
            full(Wa1.shape), full(ba1.shape),
            full(Wa2.shape), full(ba2.shape),
            full(Wa3.shape), full(ba3.shape),
        ],
        out_specs=pl.BlockSpec((BB, D), lambda i: (i, 0)),
        out_shape=jax.ShapeDtypeStruct((B, D), jnp.float32),
        compiler_params=pltpu.CompilerParams(
            dimension_semantics=("parallel",),
        ),
    )(x1p2, half_rows, rating_rows, urep, opinion,
      W1, b1, W2, b2, Wa1, ba1, Wa2, ba2, Wa3, ba3)


def kernel(nodes, user_item_pair, rating, item_table, user_table,
           opinion_table, W1, b1, W2, b2, Wa1, ba1, Wa2, ba2, Wa3, ba3):
    B, L = user_item_pair.shape
    D = item_table.shape[1]

    uidx = nodes.astype(jnp.int32)
    urep = _sc_user_gather(user_table.T, uidx)

    g2 = _tc_transform(item_table.T, W1[:, :D])
    iidx = user_item_pair.reshape(B * L).astype(jnp.int32)
    x1p2 = _sc_row_gather(g2, iidx % _H)
    half_rows = (iidx // _H).reshape(B * L, 1)

    rating_rows = rating.reshape(B * L, 1).astype(jnp.int32)
    return _tc_forward(
        x1p2, half_rows, rating_rows, urep, opinion_table,
        W1, b1.reshape(1, D), W2, b2.reshape(1, D),
        Wa1, ba1.reshape(1, D), Wa2, ba2.reshape(1, D),
        Wa3, ba3.reshape(1, 1),
    )


# speedup vs baseline: 3.3962x; 1.1156x over previous
import functools

import jax
import jax.numpy as jnp
from jax import lax
from jax.experimental import pallas as pl
from jax.experimental.pallas import tpu as pltpu
from jax.experimental.pallas import tpu_sc as plsc

_H = 262144


def _tc_transform(table_t, w1h, nb=8192):
    D, V = table_t.shape
    grid = (_H // nb,)
    shift = _H // nb
    n_hi_max = V // nb

    def body(b0_ref, b1_ref, b2_ref, b3_ref, w_ref, out_ref):
        w = w_ref[...]

        def gt(ref):
            g = lax.dot_general(ref[...], w, (((0,), (1,)), ((), ())),
                                preferred_element_type=jnp.float32)
            return g.astype(jnp.bfloat16)

        def pack(lo, hi):
            lo_u = lax.bitcast_convert_type(lo, jnp.uint16).astype(jnp.uint32)
            hi_u = lax.bitcast_convert_type(hi, jnp.uint16).astype(jnp.uint32)
            return lax.bitcast_convert_type((hi_u << 16) | lo_u, jnp.int32)

        p01 = pack(gt(b0_ref), gt(b1_ref))
        p23 = pack(gt(b2_ref), gt(b3_ref))
        out_ref[...] = jnp.concatenate([p01, p23], axis=1)

    return pl.pallas_call(
        body,
        grid=grid,
        in_specs=[
            pl.BlockSpec((D, nb), lambda i: (0, i)),
            pl.BlockSpec((D, nb), lambda i: (0, i + shift)),
            pl.BlockSpec((D, nb),
                         lambda i: (0, jnp.minimum(i + 2 * shift, n_hi_max))),
            pl.BlockSpec((D, nb),
                         lambda i: (0, jnp.minimum(i + 3 * shift, n_hi_max))),
            pl.BlockSpec((D, D), lambda i: (0, 0)),
        ],
        out_specs=pl.BlockSpec((nb, 2 * D), lambda i: (i, 0)),
        out_shape=jax.ShapeDtypeStruct((_H, 2 * D), jnp.int32),
        compiler_params=pltpu.CompilerParams(
            dimension_semantics=("parallel",),
        ),
    )(table_t, table_t, table_t, table_t, w1h)


def _sc_row_gather(g2, iidx):
    H, W = g2.shape
    BL = iidx.shape[0]

    info = plsc.get_sparse_core_info()
    nw = info.num_cores * info.num_subcores
    nc = info.num_cores
    ipw = BL // nw
    assert BL % nw == 0 and ipw % 16 == 0
    HALF = ipw // 2
    CH = 80
    n_ch = HALF // CH
    assert HALF % CH == 0

    mesh = plsc.VectorSubcoreMesh(core_axis_name="c", subcore_axis_name="s")

    @functools.partial(
        pl.kernel,
        mesh=mesh,
        out_type=jax.ShapeDtypeStruct((BL, W), jnp.int32),
        scratch_types=[
            pltpu.VMEM((ipw,), jnp.int32),
            pltpu.VMEM((HALF, W), jnp.int32),
            pltpu.SemaphoreType.DMA,
        ],
    )
    def gather_k(g_hbm, iidx_hbm, hist_out, iidx_v, rows_v, sem):
        wid = lax.axis_index("s") * nc + lax.axis_index("c")
        base = wid * ipw

        pltpu.sync_copy(iidx_hbm.at[pl.ds(base, ipw)], iidx_v)

        for h in range(2):
            def fire(c, carry):
                pltpu.make_async_copy(
                    g_hbm.at[iidx_v.at[pl.ds(h * HALF + c * CH, CH)]],
                    rows_v.at[pl.ds(c * CH, CH)],
                    sem,
                ).start()
                return carry

            lax.fori_loop(0, n_ch, fire, 0)

            def drain(c, carry):
                pltpu.make_async_copy(
                    g_hbm.at[iidx_v.at[pl.ds(0, CH)]],
                    rows_v.at[pl.ds(0, CH)],
                    sem,
                ).wait()
                return carry

            lax.fori_loop(0, n_ch, drain, 0)
            pltpu.sync_copy(rows_v, hist_out.at[pl.ds(base + h * HALF, HALF)])

    return gather_k(g2, iidx)


def _sc_user_gather(user_t, uidx):
    D, V = user_t.shape
    B = uidx.shape[0]

    info = plsc.get_sparse_core_info()
    nw = info.num_cores * info.num_subcores
    nc = info.num_cores
    CH = B // nw
    assert B % nw == 0

    mesh = plsc.VectorSubcoreMesh(core_axis_name="c", subcore_axis_name="s")

    @functools.partial(
        pl.kernel,
        mesh=mesh,
        out_type=jax.ShapeDtypeStruct((B * D,), jnp.float32),
        scratch_types=[
            pltpu.VMEM((CH,), jnp.int32),
            pltpu.VMEM((D, 128), jnp.float32),
            pltpu.VMEM((CH * D,), jnp.float32),
            pltpu.SemaphoreType.DMA,
        ],
        compiler_params=pltpu.CompilerParams(needs_layout_passes=False),
    )
    def user_k(user_hbm, uidx_hbm, out_hbm, idx_v, blk_v, stage_v, sem):
        wid = lax.axis_index("s") * nc + lax.axis_index("c")

        pltpu.sync_copy(uidx_hbm.at[pl.ds(wid * CH, CH)], idx_v)

        def one_user(j, carry):
            vec = idx_v[pl.ds((j // 16) * 16, 16)]
            lane = lax.iota(jnp.int32, 16)
            r = jnp.max(jnp.where(lane == j % 16, vec, 0))
            off = pl.multiple_of((r // 128) * 128, 128)
            rl = r - off
            cp = pltpu.make_async_copy(
                user_hbm.at[:, pl.ds(off, 128)], blk_v, sem,
            )
            cp.start()
            cp.wait()
            for k in range(D // 16):
                rows = lax.iota(jnp.int32, 16) + (16 * k)
                cols = jnp.full((16,), rl, jnp.int32)
                vals = plsc.load_gather(blk_v, [rows, cols])
                pos = lax.iota(jnp.int32, 16) + (j * D + 16 * k)
                plsc.store_scatter(stage_v, [pos], vals)
            return carry

        lax.fori_loop(0, CH, one_user, 0)
        pltpu.sync_copy(stage_v, out_hbm.at[pl.ds(wid * CH * D, CH * D)])

    return user_k(user_t, uidx).reshape(B, D)


def _dot_t(x, w):
    return lax.dot_general(x, w, (((1,), (1,)), ((), ())),
                           preferred_element_type=jnp.float32)


def _dot(x, w):
    return lax.dot_general(x, w, (((1,), (0,)), ((), ())),
                           preferred_element_type=jnp.float32)


def _tc_body(L, D,
             x1p_ref, half_ref, rat_ref, urep_ref, op_ref, w1_ref, b1_ref,
             w2_ref, b2_ref, wa1_ref, ba1_ref, wa2_ref, ba2_ref, wa3_ref,
             ba3_ref, out_ref):
    x1q = x1p_ref[...]
    R = x1q.shape[0]
    BB = R // L

    q = half_ref[...]
    v64 = jnp.where(q <= 1, x1q[:, :D], x1q[:, D:])
    u = jnp.where(q % 2 == 0, v64 << 16, v64)
    u = lax.bitwise_and(u, jnp.int32(-65536))
    x1p = lax.bitcast_convert_type(u, jnp.float32)

    w1 = w1_ref[...]
    r1 = _dot_t(op_ref[...], w1[:, D:])

    rat = rat_ref[...]
    r_c = jnp.zeros_like(x1p)
    for r in range(r1.shape[0]):
        m = (rat == r).astype(jnp.float32)
        r_c = r_c + m * r1[r:r + 1, :]
    x1 = jnp.maximum(x1p + r_c + b1_ref[...], 0.0)

    o = jnp.maximum(_dot_t(x1, w2_ref[...]) + b2_ref[...], 0.0)

    wa1 = wa1_ref[...]
    u_c = _dot_t(urep_ref[...], wa1[:, D:])
    rows = lax.broadcasted_iota(jnp.int32, (R, BB), 0) // L
    cols = lax.broadcasted_iota(jnp.int32, (R, BB), 1)
    rep = (rows == cols).astype(jnp.float32)
    a1 = jnp.maximum(_dot_t(o, wa1[:, :D]) + _dot(rep, u_c) + ba1_ref[...], 0.0)
    a2 = jnp.maximum(_dot_t(a1, wa2_ref[...]) + ba2_ref[...], 0.0)
    s = jnp.sum(a2 * wa3_ref[...], axis=1, keepdims=True) + ba3_ref[...]

    e = jnp.exp(s - jnp.max(s))
    srows = lax.broadcasted_iota(jnp.int32, (BB, R), 0)
    scols = lax.broadcasted_iota(jnp.int32, (BB, R), 1) // L
    seg = (srows == scols).astype(jnp.float32)
    num = _dot(seg, o * e)
    den = _dot(seg, e)
    out_ref[...] = num / den


def _tc_forward(x1p2, half_rows, rating_rows, urep, opinion,
                W1, b1, W2, b2, Wa1, ba1, Wa2, ba2, Wa3, ba3, BB=128):
    B, D = urep.shape
    L = x1p2.shape[0] // B
    grid = (B // BB,)
    R = BB * L

    def full(shape):
        return pl.BlockSpec(shape, lambda i: (0, 0))

    return pl.pallas_call(
        functools.partial(_tc_body, L, D),
        grid=grid,
        in_specs=[
            pl.BlockSpec((R, 2 * D), lambda i: (i, 0)),
            pl.BlockSpec((R, 1), lambda i: (i, 0)),
            pl.BlockSpec((R, 1), lambda i: (i, 0)),
            pl.BlockSpec((BB, D), lambda i: (i, 0)),
            full(opinion.shape),
            full(W1.shape), full(b1.shape),
            full(W2.shape), full(b2.shape),
            full(Wa1.shape), full(ba1.shape),
            full(Wa2.shape), full(ba2.shape),
            full(Wa3.shape), full(ba3.shape),
        ],
        out_specs=pl.BlockSpec((BB, D), lambda i: (i, 0)),
        out_shape=jax.ShapeDtypeStruct((B, D), jnp.float32),
        compiler_params=pltpu.CompilerParams(
            dimension_semantics=("parallel",),
        ),
    )(x1p2, half_rows, rating_rows, urep, opinion,
      W1, b1, W2, b2, Wa1, ba1, Wa2, ba2, Wa3, ba3)


def kernel(nodes, user_item_pair, rating, item_table, user_table,
           opinion_table, W1, b1, W2, b2, Wa1, ba1, Wa2, ba2, Wa3, ba3):
    B, L = user_item_pair.shape
    D = item_table.shape[1]

    uidx = nodes.astype(jnp.int32)
    urep = _sc_user_gather(user_table.T, uidx)

    g2 = _tc_transform(item_table.T, W1[:, :D])
    iidx = user_item_pair.reshape(B * L).astype(jnp.int32)
    x1p2 = _sc_row_gather(g2, iidx % _H)
    half_rows = (iidx // _H).reshape(B * L, 1)

    rating_rows = rating.reshape(B * L, 1).astype(jnp.int32)
    return _tc_forward(
        x1p2, half_rows, rating_rows, urep, opinion_table,
        W1, b1.reshape(1, D), W2, b2.reshape(1, D),
        Wa1, ba1.reshape(1, D), Wa2, ba2.reshape(1, D),
        Wa3, ba3.reshape(1, 1),
    )


# speedup vs baseline: 3.7782x; 1.1125x over previous
import functools

import jax
import jax.numpy as jnp
from jax import lax
from jax.experimental import pallas as pl
from jax.experimental.pallas import tpu as pltpu
from jax.experimental.pallas import tpu_sc as plsc

_H = 262144


def _tc_transform(table_t, w1h, nb=8192):
    D, V = table_t.shape
    grid = (_H // nb,)
    shift = _H // nb
    n_hi_max = V // nb

    def body(b0_ref, b1_ref, b2_ref, b3_ref, w_ref, out_ref):
        w = w_ref[...].astype(jnp.bfloat16)

        def gt(ref):
            g = lax.dot_general(ref[...].astype(jnp.bfloat16), w,
                                (((0,), (1,)), ((), ())),
                                preferred_element_type=jnp.float32)
            return g.astype(jnp.bfloat16)

        def pack(lo, hi):
            lo_u = lax.bitcast_convert_type(lo, jnp.uint16).astype(jnp.uint32)
            hi_u = lax.bitcast_convert_type(hi, jnp.uint16).astype(jnp.uint32)
            return lax.bitcast_convert_type((hi_u << 16) | lo_u, jnp.int32)

        p01 = pack(gt(b0_ref), gt(b1_ref))
        p23 = pack(gt(b2_ref), gt(b3_ref))
        out_ref[...] = jnp.concatenate([p01, p23], axis=1)

    return pl.pallas_call(
        body,
        grid=grid,
        in_specs=[
            pl.BlockSpec((D, nb), lambda i: (0, i)),
            pl.BlockSpec((D, nb), lambda i: (0, i + shift)),
            pl.BlockSpec((D, nb),
                         lambda i: (0, jnp.minimum(i + 2 * shift, n_hi_max))),
            pl.BlockSpec((D, nb),
                         lambda i: (0, jnp.minimum(i + 3 * shift, n_hi_max))),
            pl.BlockSpec((D, D), lambda i: (0, 0)),
        ],
        out_specs=pl.BlockSpec((nb, 2 * D), lambda i: (i, 0)),
        out_shape=jax.ShapeDtypeStruct((_H, 2 * D), jnp.int32),
        compiler_params=pltpu.CompilerParams(
            dimension_semantics=("parallel",),
        ),
    )(table_t, table_t, table_t, table_t, w1h)


def _sc_row_gather(g2, iidx):
    H, W = g2.shape
    BL = iidx.shape[0]

    info = plsc.get_sparse_core_info()
    nw = info.num_cores * info.num_subcores
    nc = info.num_cores
    ipw = BL // nw
    assert BL % nw == 0 and ipw % 16 == 0
    HALF = ipw // 2
    CH = 80
    n_ch = HALF // CH
    assert HALF % CH == 0

    mesh = plsc.VectorSubcoreMesh(core_axis_name="c", subcore_axis_name="s")

    @functools.partial(
        pl.kernel,
        mesh=mesh,
        out_type=jax.ShapeDtypeStruct((BL, W), jnp.int32),
        scratch_types=[
            pltpu.VMEM((ipw,), jnp.int32),
            pltpu.VMEM((HALF, W), jnp.int32),
            pltpu.SemaphoreType.DMA,
        ],
    )
    def gather_k(g_hbm, iidx_hbm, hist_out, iidx_v, rows_v, sem):
        wid = lax.axis_index("s") * nc + lax.axis_index("c")
        base = wid * ipw

        pltpu.sync_copy(iidx_hbm.at[pl.ds(base, ipw)], iidx_v)

        for h in range(2):
            def fire(c, carry):
                pltpu.make_async_copy(
                    g_hbm.at[iidx_v.at[pl.ds(h * HALF + c * CH, CH)]],
                    rows_v.at[pl.ds(c * CH, CH)],
                    sem,
                ).start()
                return carry

            lax.fori_loop(0, n_ch, fire, 0)

            def drain(c, carry):
                pltpu.make_async_copy(
                    g_hbm.at[iidx_v.at[pl.ds(0, CH)]],
                    rows_v.at[pl.ds(0, CH)],
                    sem,
                ).wait()
                return carry

            lax.fori_loop(0, n_ch, drain, 0)
            pltpu.sync_copy(rows_v, hist_out.at[pl.ds(base + h * HALF, HALF)])

    return gather_k(g2, iidx)


def _sc_user_gather(user_t, uidx):
    D, V = user_t.shape
    B = uidx.shape[0]

    info = plsc.get_sparse_core_info()
    nw = info.num_cores * info.num_subcores
    nc = info.num_cores
    CH = B // nw
    assert B % nw == 0

    mesh = plsc.VectorSubcoreMesh(core_axis_name="c", subcore_axis_name="s")

    @functools.partial(
        pl.kernel,
        mesh=mesh,
        out_type=jax.ShapeDtypeStruct((B * D,), jnp.float32),
        scratch_types=[
            pltpu.VMEM((CH,), jnp.int32),
            pltpu.VMEM((D, 128), jnp.float32),
            pltpu.VMEM((CH * D,), jnp.float32),
            pltpu.SemaphoreType.DMA,
        ],
        compiler_params=pltpu.CompilerParams(needs_layout_passes=False),
    )
    def user_k(user_hbm, uidx_hbm, out_hbm, idx_v, blk_v, stage_v, sem):
        wid = lax.axis_index("s") * nc + lax.axis_index("c")

        pltpu.sync_copy(uidx_hbm.at[pl.ds(wid * CH, CH)], idx_v)

        def one_user(j, carry):
            vec = idx_v[pl.ds((j // 16) * 16, 16)]
            lane = lax.iota(jnp.int32, 16)
            r = jnp.max(jnp.where(lane == j % 16, vec, 0))
            off = pl.multiple_of((r // 128) * 128, 128)
            rl = r - off
            cp = pltpu.make_async_copy(
                user_hbm.at[:, pl.ds(off, 128)], blk_v, sem,
            )
            cp.start()
            cp.wait()
            for k in range(D // 16):
                rows = lax.iota(jnp.int32, 16) + (16 * k)
                cols = jnp.full((16,), rl, jnp.int32)
                vals = plsc.load_gather(blk_v, [rows, cols])
                pos = lax.iota(jnp.int32, 16) + (j * D + 16 * k)
                plsc.store_scatter(stage_v, [pos], vals)
            return carry

        lax.fori_loop(0, CH, one_user, 0)
        pltpu.sync_copy(stage_v, out_hbm.at[pl.ds(wid * CH * D, CH * D)])

    return user_k(user_t, uidx).reshape(B, D)


def _dot_t(x, w):
    return lax.dot_general(x, w, (((1,), (1,)), ((), ())),
                           preferred_element_type=jnp.float32)


def _dot(x, w):
    return lax.dot_general(x, w, (((1,), (0,)), ((), ())),
                           preferred_element_type=jnp.float32)


def _tc_body(L, D,
             x1p_ref, half_ref, rat_ref, urep_ref, op_ref, w1_ref, b1_ref,
             w2_ref, b2_ref, wa1_ref, ba1_ref, wa2_ref, ba2_ref, wa3_ref,
             ba3_ref, out_ref):
    x1q = x1p_ref[...]
    R = x1q.shape[0]
    BB = R // L

    q = half_ref[...]
    v64 = jnp.where(q <= 1, x1q[:, :D], x1q[:, D:])
    u = jnp.where(q % 2 == 0, v64 << 16, v64)
    u = lax.bitwise_and(u, jnp.int32(-65536))
    x1p = lax.bitcast_convert_type(u, jnp.float32)

    w1 = w1_ref[...]
    r1 = _dot_t(op_ref[...], w1[:, D:])

    rat = rat_ref[...]
    r_c = jnp.zeros_like(x1p)
    for r in range(r1.shape[0]):
        m = (rat == r).astype(jnp.float32)
        r_c = r_c + m * r1[r:r + 1, :]
    x1 = jnp.maximum(x1p + r_c + b1_ref[...], 0.0)

    o = jnp.maximum(_dot_t(x1, w2_ref[...]) + b2_ref[...], 0.0)

    wa1 = wa1_ref[...]
    u_c = _dot_t(urep_ref[...], wa1[:, D:])
    rows = lax.broadcasted_iota(jnp.int32, (R, BB), 0) // L
    cols = lax.broadcasted_iota(jnp.int32, (R, BB), 1)
    rep = (rows == cols).astype(jnp.float32)
    a1 = jnp.maximum(_dot_t(o, wa1[:, :D]) + _dot(rep, u_c) + ba1_ref[...], 0.0)
    a2 = jnp.maximum(_dot_t(a1, wa2_ref[...]) + ba2_ref[...], 0.0)
    s = jnp.sum(a2 * wa3_ref[...], axis=1, keepdims=True) + ba3_ref[...]

    e = jnp.exp(s - jnp.max(s))
    srows = lax.broadcasted_iota(jnp.int32, (BB, R), 0)
    scols = lax.broadcasted_iota(jnp.int32, (BB, R), 1) // L
    seg = (srows == scols).astype(jnp.float32)
    num = _dot(seg, o * e)
    den = _dot(seg, e)
    out_ref[...] = num / den


def _tc_forward(x1p2, half_rows, rating_rows, urep, opinion,
                W1, b1, W2, b2, Wa1, ba1, Wa2, ba2, Wa3, ba3, BB=128):
    B, D = urep.shape
    L = x1p2.shape[0] // B
    grid = (B // BB,)
    R = BB * L

    def full(shape):
        return pl.BlockSpec(shape, lambda i: (0, 0))

    return pl.pallas_call(
        functools.partial(_tc_body, L, D),
        grid=grid,
        in_specs=[
            pl.BlockSpec((R, 2 * D), lambda i: (i, 0)),
            pl.BlockSpec((R, 1), lambda i: (i, 0)),
            pl.BlockSpec((R, 1), lambda i: (i, 0)),
            pl.BlockSpec((BB, D), lambda i: (i, 0)),
            full(opinion.shape),
            full(W1.shape), full(b1.shape),
            full(W2.shape), full(b2.shape),
            full(Wa1.shape), full(ba1.shape),
            full(Wa2.shape), full(ba2.shape),
            full(Wa3.shape), full(ba3.shape),
        ],
        out_specs=pl.BlockSpec((BB, D), lambda i: (i, 0)),
        out_shape=jax.ShapeDtypeStruct((B, D), jnp.float32),
        compiler_params=pltpu.CompilerParams(
            dimension_semantics=("parallel",),
        ),
    )(x1p2, half_rows, rating_rows, urep, opinion,
      W1, b1, W2, b2, Wa1, ba1, Wa2, ba2, Wa3, ba3)


def kernel(nodes, user_item_pair, rating, item_table, user_table,
           opinion_table, W1, b1, W2, b2, Wa1, ba1, Wa2, ba2, Wa3, ba3):
    B, L = user_item_pair.shape
    D = item_table.shape[1]

    uidx = nodes.astype(jnp.int32)
    urep = _sc_user_gather(user_table.T, uidx)

    g2 = _tc_transform(item_table.T, W1[:, :D])
    iidx = user_item_pair.reshape(B * L).astype(jnp.int32)
    x1p2 = _sc_row_gather(g2, iidx % _H)
    half_rows = (iidx // _H).reshape(B * L, 1)

    rating_rows = rating.reshape(B * L, 1).astype(jnp.int32)
    return _tc_forward(
        x1p2, half_rows, rating_rows, urep, opinion_table,
        W1, b1.reshape(1, D), W2, b2.reshape(1, D),
        Wa1, ba1.reshape(1, D), Wa2, ba2.reshape(1, D),
        Wa3, ba3.reshape(1, 1),
    )


# speedup vs baseline: 3.9597x; 1.0480x over previous
import functools

import jax
import jax.numpy as jnp
from jax import lax
from jax.experimental import pallas as pl
from jax.experimental.pallas import tpu as pltpu
from jax.experimental.pallas import tpu_sc as plsc

_H = 262144


def _tc_transform(table_t, w1h, nb=8192):
    D, V = table_t.shape
    grid = (_H // nb,)
    shift = _H // nb
    n_hi_max = V // nb

    def body(b0_ref, b1_ref, b2_ref, b3_ref, w_ref, out_ref):
        w = w_ref[...].astype(jnp.bfloat16)

        def gt(ref):
            g = lax.dot_general(ref[...].astype(jnp.bfloat16), w,
                                (((0,), (1,)), ((), ())),
                                preferred_element_type=jnp.float32)
            return g.astype(jnp.bfloat16)

        def pack(lo, hi):
            lo_u = lax.bitcast_convert_type(lo, jnp.uint16).astype(jnp.uint32)
            hi_u = lax.bitcast_convert_type(hi, jnp.uint16).astype(jnp.uint32)
            return lax.bitcast_convert_type((hi_u << 16) | lo_u, jnp.int32)

        p01 = pack(gt(b0_ref), gt(b1_ref))
        p23 = pack(gt(b2_ref), gt(b3_ref))
        out_ref[...] = jnp.concatenate([p01, p23], axis=1)

    return pl.pallas_call(
        body,
        grid=grid,
        in_specs=[
            pl.BlockSpec((D, nb), lambda i: (0, i)),
            pl.BlockSpec((D, nb), lambda i: (0, i + shift)),
            pl.BlockSpec((D, nb),
                         lambda i: (0, jnp.minimum(i + 2 * shift, n_hi_max))),
            pl.BlockSpec((D, nb),
                         lambda i: (0, jnp.minimum(i + 3 * shift, n_hi_max))),
            pl.BlockSpec((D, D), lambda i: (0, 0)),
        ],
        out_specs=pl.BlockSpec((nb, 2 * D), lambda i: (i, 0)),
        out_shape=jax.ShapeDtypeStruct((_H, 2 * D), jnp.int32),
        compiler_params=pltpu.CompilerParams(
            dimension_semantics=("parallel",),
        ),
    )(table_t, table_t, table_t, table_t, w1h)


def _sc_row_gather(g2, iidx):
    H, W = g2.shape
    BL = iidx.shape[0]

    info = plsc.get_sparse_core_info()
    nw = info.num_cores * info.num_subcores
    nc = info.num_cores
    ipw = BL // nw
    assert BL % nw == 0 and ipw % 16 == 0
    HALF = ipw // 2
    CH = 80
    n_ch = HALF // CH
    assert HALF % CH == 0

    mesh = plsc.VectorSubcoreMesh(core_axis_name="c", subcore_axis_name="s")

    @functools.partial(
        pl.kernel,
        mesh=mesh,
        out_type=jax.ShapeDtypeStruct((BL, W), jnp.int32),
        scratch_types=[
            pltpu.VMEM((ipw,), jnp.int32),
            pltpu.VMEM((HALF, W), jnp.int32),
            pltpu.SemaphoreType.DMA,
        ],
    )
    def gather_k(g_hbm, iidx_hbm, hist_out, iidx_v, rows_v, sem):
        wid = lax.axis_index("s") * nc + lax.axis_index("c")
        base = wid * ipw

        pltpu.sync_copy(iidx_hbm.at[pl.ds(base, ipw)], iidx_v)

        for h in range(2):
            def fire(c, carry):
                pltpu.make_async_copy(
                    g_hbm.at[iidx_v.at[pl.ds(h * HALF + c * CH, CH)]],
                    rows_v.at[pl.ds(c * CH, CH)],
                    sem,
                ).start()
                return carry

            lax.fori_loop(0, n_ch, fire, 0)

            def drain(c, carry):
                pltpu.make_async_copy(
                    g_hbm.at[iidx_v.at[pl.ds(0, CH)]],
                    rows_v.at[pl.ds(0, CH)],
                    sem,
                ).wait()
                return carry

            lax.fori_loop(0, n_ch, drain, 0)
            pltpu.sync_copy(rows_v, hist_out.at[pl.ds(base + h * HALF, HALF)])

    return gather_k(g2, iidx)


def _sc_user_gather(user_t, uidx):
    D, V = user_t.shape
    B = uidx.shape[0]

    info = plsc.get_sparse_core_info()
    nw = info.num_cores * info.num_subcores
    nc = info.num_cores
    CH = B // nw
    assert B % nw == 0

    mesh = plsc.VectorSubcoreMesh(core_axis_name="c", subcore_axis_name="s")

    @functools.partial(
        pl.kernel,
        mesh=mesh,
        out_type=jax.ShapeDtypeStruct((B * D,), jnp.float32),
        scratch_types=[
            pltpu.VMEM((CH,), jnp.int32),
            pltpu.VMEM((D, 128), jnp.float32),
            pltpu.VMEM((CH * D,), jnp.float32),
            pltpu.SemaphoreType.DMA,
        ],
        compiler_params=pltpu.CompilerParams(needs_layout_passes=False),
    )
    def user_k(user_hbm, uidx_hbm, out_hbm, idx_v, blk_v, stage_v, sem):
        wid = lax.axis_index("s") * nc + lax.axis_index("c")

        pltpu.sync_copy(uidx_hbm.at[pl.ds(wid * CH, CH)], idx_v)

        def one_user(j, carry):
            vec = idx_v[pl.ds((j // 16) * 16, 16)]
            lane = lax.iota(jnp.int32, 16)
            r = jnp.max(jnp.where(lane == j % 16, vec, 0))
            off = pl.multiple_of((r // 128) * 128, 128)
            rl = r - off
            cp = pltpu.make_async_copy(
                user_hbm.at[:, pl.ds(off, 128)], blk_v, sem,
            )
            cp.start()
            cp.wait()
            for k in range(D // 16):
                rows = lax.iota(jnp.int32, 16) + (16 * k)
                cols = jnp.full((16,), rl, jnp.int32)
                vals = plsc.load_gather(blk_v, [rows, cols])
                pos = lax.iota(jnp.int32, 16) + (j * D + 16 * k)
                plsc.store_scatter(stage_v, [pos], vals)
            return carry

        lax.fori_loop(0, CH, one_user, 0)
        pltpu.sync_copy(stage_v, out_hbm.at[pl.ds(wid * CH * D, CH * D)])

    return user_k(user_t, uidx).reshape(B, D)


def _dot_t(x, w):
    return lax.dot_general(x, w, (((1,), (1,)), ((), ())),
                           preferred_element_type=jnp.float32)


def _dot(x, w):
    return lax.dot_general(x, w, (((1,), (0,)), ((), ())),
                           preferred_element_type=jnp.float32)


def _tc_body(L, D,
             x1p_ref, half_ref, rat_ref, urep_ref, op_ref, w1_ref, b1_ref,
             w2_ref, b2_ref, wa1_ref, ba1_ref, wa2_ref, ba2_ref, wa3_ref,
             ba3_ref, out_ref):
    x1q = x1p_ref[...]
    R = x1q.shape[0]
    BB = R // L

    q = half_ref[...]
    v64 = jnp.where(q <= 1, x1q[:, :D], x1q[:, D:])
    u = jnp.where(q % 2 == 0, v64 << 16, v64)
    u = lax.bitwise_and(u, jnp.int32(-65536))
    x1p = lax.bitcast_convert_type(u, jnp.float32)

    w1 = w1_ref[...]
    r1 = _dot_t(op_ref[...], w1[:, D:])

    rat = rat_ref[...]
    onehot = (rat == lax.broadcasted_iota(jnp.int32, (1, 8), 1))
    r1p = jnp.concatenate(
        [r1, jnp.zeros((8 - r1.shape[0], D), jnp.float32)], axis=0)
    r_c = _dot(onehot.astype(jnp.float32), r1p)
    x1 = jnp.maximum(x1p + r_c + b1_ref[...], 0.0)

    o = jnp.maximum(_dot_t(x1, w2_ref[...]) + b2_ref[...], 0.0)

    wa1 = wa1_ref[...]
    u_c = _dot_t(urep_ref[...], wa1[:, D:])
    rows = lax.broadcasted_iota(jnp.int32, (R, BB), 0) // L
    cols = lax.broadcasted_iota(jnp.int32, (R, BB), 1)
    rep = (rows == cols).astype(jnp.float32)
    a1 = jnp.maximum(_dot_t(o, wa1[:, :D]) + _dot(rep, u_c) + ba1_ref[...], 0.0)
    a2 = jnp.maximum(_dot_t(a1, wa2_ref[...]) + ba2_ref[...], 0.0)
    s = jnp.sum(a2 * wa3_ref[...], axis=1, keepdims=True) + ba3_ref[...]

    e = jnp.exp(s - jnp.max(s))
    srows = lax.broadcasted_iota(jnp.int32, (BB, R), 0)
    scols = lax.broadcasted_iota(jnp.int32, (BB, R), 1) // L
    seg = (srows == scols).astype(jnp.float32)
    num = _dot(seg, o * e)
    den = _dot(seg, e)
    out_ref[...] = num / den


def _tc_forward(x1p2, half_rows, rating_rows, urep, opinion,
                W1, b1, W2, b2, Wa1, ba1, Wa2, ba2, Wa3, ba3, BB=128):
    B, D = urep.shape
    L = x1p2.shape[0] // B
    grid = (B // BB,)
    R = BB * L

    def full(shape):
        return pl.BlockSpec(shape, lambda i: (0, 0))

    return pl.pallas_call(
        functools.partial(_tc_body, L, D),
        grid=grid,
        in_specs=[
            pl.BlockSpec((R, 2 * D), lambda i: (i, 0)),
            pl.BlockSpec((R, 1), lambda i: (i, 0)),
            pl.BlockSpec((R, 1), lambda i: (i, 0)),
            pl.BlockSpec((BB, D), lambda i: (i, 0)),
            full(opinion.shape),
            full(W1.shape), full(b1.shape),
            full(W2.shape), full(b2.shape),
            full(Wa1.shape), full(ba1.shape),
            full(Wa2.shape), full(ba2.shape),
            full(Wa3.shape), full(ba3.shape),
        ],
        out_specs=pl.BlockSpec((BB, D), lambda i: (i, 0)),
        out_shape=jax.ShapeDtypeStruct((B, D), jnp.float32),
        compiler_params=pltpu.CompilerParams(
            dimension_semantics=("parallel",),
        ),
    )(x1p2, half_rows, rating_rows, urep, opinion,
      W1, b1, W2, b2, Wa1, ba1, Wa2, ba2, Wa3, ba3)


def kernel(nodes, user_item_pair, rating, item_table, user_table,
           opinion_table, W1, b1, W2, b2, Wa1, ba1, Wa2, ba2, Wa3, ba3):
    B, L = user_item_pair.shape
    D = item_table.shape[1]

    uidx = nodes.astype(jnp.int32)
    urep = _sc_user_gather(user_table.T, uidx)

    g2 = _tc_transform(item_table.T, W1[:, :D])
    iidx = user_item_pair.reshape(B * L).astype(jnp.int32)
    x1p2 = _sc_row_gather(g2, iidx % _H)
    half_rows = (iidx // _H).reshape(B * L, 1)

    rating_rows = rating.reshape(B * L, 1).astype(jnp.int32)
    return _tc_forward(
        x1p2, half_rows, rating_rows, urep, opinion_table,
        W1, b1.reshape(1, D), W2, b2.reshape(1, D),
        Wa1, ba1.reshape(1, D), Wa2, ba2.reshape(1, D),
        Wa3, ba3.reshape(1, 1),
    )


# speedup vs baseline: 3.9747x; 1.0038x over previous
import functools

import jax
import jax.numpy as jnp
from jax import lax
from jax.experimental import pallas as pl
from jax.experimental.pallas import tpu as pltpu
from jax.experimental.pallas import tpu_sc as plsc

_H = 262144


def _tc_transform(table_t, w1h, nb=8192):
    D, V = table_t.shape
    grid = (_H // nb,)
    shift = _H // nb
    n_hi_max = V // nb

    def body(b0_ref, b1_ref, b2_ref, b3_ref, w_ref, out_ref):
        w = w_ref[...].astype(jnp.bfloat16)

        def gt(ref):
            g = lax.dot_general(ref[...].astype(jnp.bfloat16), w,
                                (((0,), (1,)), ((), ())),
                                preferred_element_type=jnp.float32)
            return g.astype(jnp.bfloat16)

        def pack(lo, hi):
            lo_u = lax.bitcast_convert_type(lo, jnp.uint16).astype(jnp.uint32)
            hi_u = lax.bitcast_convert_type(hi, jnp.uint16).astype(jnp.uint32)
            return lax.bitcast_convert_type((hi_u << 16) | lo_u, jnp.int32)

        p01 = pack(gt(b0_ref), gt(b1_ref))
        p23 = pack(gt(b2_ref), gt(b3_ref))
        out_ref[...] = jnp.concatenate([p01, p23], axis=1)

    return pl.pallas_call(
        body,
        grid=grid,
        in_specs=[
            pl.BlockSpec((D, nb), lambda i: (0, i)),
            pl.BlockSpec((D, nb), lambda i: (0, i + shift)),
            pl.BlockSpec((D, nb),
                         lambda i: (0, jnp.minimum(i + 2 * shift, n_hi_max))),
            pl.BlockSpec((D, nb),
                         lambda i: (0, jnp.minimum(i + 3 * shift, n_hi_max))),
            pl.BlockSpec((D, D), lambda i: (0, 0)),
        ],
        out_specs=pl.BlockSpec((nb, 2 * D), lambda i: (i, 0)),
        out_shape=jax.ShapeDtypeStruct((_H, 2 * D), jnp.int32),
        compiler_params=pltpu.CompilerParams(
            dimension_semantics=("parallel",),
        ),
    )(table_t, table_t, table_t, table_t, w1h)


def _sc_row_gather(g2, iidx):
    H, W = g2.shape
    BL = iidx.shape[0]

    info = plsc.get_sparse_core_info()
    nw = info.num_cores * info.num_subcores
    nc = info.num_cores
    ipw = BL // nw
    assert BL % nw == 0 and ipw % 16 == 0
    HALF = ipw // 2
    CH = 80
    n_ch = HALF // CH
    assert HALF % CH == 0

    mesh = plsc.VectorSubcoreMesh(core_axis_name="c", subcore_axis_name="s")

    @functools.partial(
        pl.kernel,
        mesh=mesh,
        out_type=jax.ShapeDtypeStruct((BL, W), jnp.int32),
        scratch_types=[
            pltpu.VMEM((ipw,), jnp.int32),
            pltpu.VMEM((HALF, W), jnp.int32),
            pltpu.SemaphoreType.DMA,
        ],
    )
    def gather_k(g_hbm, iidx_hbm, hist_out, iidx_v, rows_v, sem):
        wid = lax.axis_index("s") * nc + lax.axis_index("c")
        base = wid * ipw

        pltpu.sync_copy(iidx_hbm.at[pl.ds(base, ipw)], iidx_v)

        for h in range(2):
            def fire(c, carry):
                pltpu.make_async_copy(
                    g_hbm.at[iidx_v.at[pl.ds(h * HALF + c * CH, CH)]],
                    rows_v.at[pl.ds(c * CH, CH)],
                    sem,
                ).start()
                return carry

            lax.fori_loop(0, n_ch, fire, 0)

            def drain(c, carry):
                pltpu.make_async_copy(
                    g_hbm.at[iidx_v.at[pl.ds(0, CH)]],
                    rows_v.at[pl.ds(0, CH)],
                    sem,
                ).wait()
                return carry

            lax.fori_loop(0, n_ch, drain, 0)
            pltpu.sync_copy(rows_v, hist_out.at[pl.ds(base + h * HALF, HALF)])

    return gather_k(g2, iidx)


def _sc_user_gather(user_t, uidx):
    D, V = user_t.shape
    B = uidx.shape[0]

    info = plsc.get_sparse_core_info()
    nw = info.num_cores * info.num_subcores
    nc = info.num_cores
    CH = B // nw
    assert B % nw == 0

    mesh = plsc.VectorSubcoreMesh(core_axis_name="c", subcore_axis_name="s")

    @functools.partial(
        pl.kernel,
        mesh=mesh,
        out_type=jax.ShapeDtypeStruct((B * D,), jnp.float32),
        scratch_types=[
            pltpu.VMEM((CH,), jnp.int32),
            pltpu.VMEM((D, 128), jnp.float32),
            pltpu.VMEM((D, 128), jnp.float32),
            pltpu.VMEM((CH * D,), jnp.float32),
            pltpu.SemaphoreType.DMA,
            pltpu.SemaphoreType.DMA,
        ],
        compiler_params=pltpu.CompilerParams(needs_layout_passes=False),
    )
    def user_k(user_hbm, uidx_hbm, out_hbm, idx_v, blk0_v, blk1_v, stage_v,
               sem0, sem1):
        wid = lax.axis_index("s") * nc + lax.axis_index("c")

        pltpu.sync_copy(uidx_hbm.at[pl.ds(wid * CH, CH)], idx_v)
        lane = lax.iota(jnp.int32, 16)

        def extract_idx(j):
            vec = idx_v[pl.ds((j // 16) * 16, 16)]
            return jnp.max(jnp.where(lane == j % 16, vec, 0))

        def extract_col(blk, rl, j):
            for k in range(D // 16):
                rows = lax.iota(jnp.int32, 16) + (16 * k)
                cols = jnp.full((16,), rl, jnp.int32)
                vals = plsc.load_gather(blk, [rows, cols])
                pos = lax.iota(jnp.int32, 16) + (j * D + 16 * k)
                plsc.store_scatter(stage_v, [pos], vals)

        def pair(p, carry):
            j0 = p * 2
            j1 = j0 + 1
            r0 = extract_idx(j0)
            r1 = extract_idx(j1)
            off0 = pl.multiple_of((r0 // 128) * 128, 128)
            off1 = pl.multiple_of((r1 // 128) * 128, 128)
            cp0 = pltpu.make_async_copy(
                user_hbm.at[:, pl.ds(off0, 128)], blk0_v, sem0)
            cp1 = pltpu.make_async_copy(
                user_hbm.at[:, pl.ds(off1, 128)], blk1_v, sem1)
            cp0.start()
            cp1.start()
            cp0.wait()
            extract_col(blk0_v, r0 - off0, j0)
            cp1.wait()
            extract_col(blk1_v, r1 - off1, j1)
            return carry

        lax.fori_loop(0, CH // 2, pair, 0)
        pltpu.sync_copy(stage_v, out_hbm.at[pl.ds(wid * CH * D, CH * D)])

    return user_k(user_t, uidx).reshape(B, D)


def _dot_t(x, w):
    return lax.dot_general(x, w, (((1,), (1,)), ((), ())),
                           preferred_element_type=jnp.float32)


def _dot(x, w):
    return lax.dot_general(x, w, (((1,), (0,)), ((), ())),
                           preferred_element_type=jnp.float32)


def _tc_body(L, D,
             x1p_ref, half_ref, rat_ref, urep_ref, op_ref, w1_ref, b1_ref,
             w2_ref, b2_ref, wa1_ref, ba1_ref, wa2_ref, ba2_ref, wa3_ref,
             ba3_ref, out_ref):
    x1q = x1p_ref[...]
    R = x1q.shape[0]
    BB = R // L

    q = half_ref[...]
    v64 = jnp.where(q <= 1, x1q[:, :D], x1q[:, D:])
    u = jnp.where(q % 2 == 0, v64 << 16, v64)
    u = lax.bitwise_and(u, jnp.int32(-65536))
    x1p = lax.bitcast_convert_type(u, jnp.float32)

    w1 = w1_ref[...]
    r1 = _dot_t(op_ref[...], w1[:, D:])

    rat = rat_ref[...]
    onehot = (rat == lax.broadcasted_iota(jnp.int32, (1, 8), 1))
    r1p = jnp.concatenate(
        [r1, jnp.zeros((8 - r1.shape[0], D), jnp.float32)], axis=0)
    r_c = _dot(onehot.astype(jnp.float32), r1p)
    x1 = jnp.maximum(x1p + r_c + b1_ref[...], 0.0)

    o = jnp.maximum(_dot_t(x1, w2_ref[...]) + b2_ref[...], 0.0)

    wa1 = wa1_ref[...]
    u_c = _dot_t(urep_ref[...], wa1[:, D:])
    rows = lax.broadcasted_iota(jnp.int32, (R, BB), 0) // L
    cols = lax.broadcasted_iota(jnp.int32, (R, BB), 1)
    rep = (rows == cols).astype(jnp.float32)
    a1 = jnp.maximum(_dot_t(o, wa1[:, :D]) + _dot(rep, u_c) + ba1_ref[...], 0.0)
    a2 = jnp.maximum(_dot_t(a1, wa2_ref[...]) + ba2_ref[...], 0.0)
    s = jnp.sum(a2 * wa3_ref[...], axis=1, keepdims=True) + ba3_ref[...]

    e = jnp.exp(s - jnp.max(s))
    srows = lax.broadcasted_iota(jnp.int32, (BB, R), 0)
    scols = lax.broadcasted_iota(jnp.int32, (BB, R), 1) // L
    seg = (srows == scols).astype(jnp.float32)
    num = _dot(seg, o * e)
    den = _dot(seg, e)
    out_ref[...] = num / den


def _tc_forward(x1p2, half_rows, rating_rows, urep, opinion,
                W1, b1, W2, b2, Wa1, ba1, Wa2, ba2, Wa3, ba3, BB=128):
    B, D = urep.shape
    L = x1p2.shape[0] // B
    grid = (B // BB,)
    R = BB * L

    def full(shape):
        return pl.BlockSpec(shape, lambda i: (0, 0))

    return pl.pallas_call(
        functools.partial(_tc_body, L, D),
        grid=grid,
        in_specs=[
            pl.BlockSpec((R, 2 * D), lambda i: (i, 0)),
            pl.BlockSpec((R, 1), lambda i: (i, 0)),
            pl.BlockSpec((R, 1), lambda i: (i, 0)),
            pl.BlockSpec((BB, D), lambda i: (i, 0)),
            full(opinion.shape),
            full(W1.shape), full(b1.shape),
            full(W2.shape), full(b2.shape),
            full(Wa1.shape), full(ba1.shape),
            full(Wa2.shape), full(ba2.shape),
            full(Wa3.shape), full(ba3.shape),
        ],
        out_specs=pl.BlockSpec((BB, D), lambda i: (i, 0)),
        out_shape=jax.ShapeDtypeStruct((B, D), jnp.float32),
        compiler_params=pltpu.CompilerParams(
            dimension_semantics=("parallel",),
        ),
    )(x1p2, half_rows, rating_rows, urep, opinion,
      W1, b1, W2, b2, Wa1, ba1, Wa2, ba2, Wa3, ba3)


def kernel(nodes, user_item_pair, rating, item_table, user_table,
           opinion_table, W1, b1, W2, b2, Wa1, ba1, Wa2, ba2, Wa3, ba3):
    B, L = user_item_pair.shape
    D = item_table.shape[1]

    uidx = nodes.astype(jnp.int32)
    urep = _sc_user_gather(user_table.T, uidx)

    g2 = _tc_transform(item_table.T, W1[:, :D])
    iidx = user_item_pair.reshape(B * L).astype(jnp.int32)
    x1p2 = _sc_row_gather(g2, iidx % _H)
    half_rows = (iidx // _H).reshape(B * L, 1)

    rating_rows = rating.reshape(B * L, 1).astype(jnp.int32)
    return _tc_forward(
        x1p2, half_rows, rating_rows, urep, opinion_table,
        W1, b1.reshape(1, D), W2, b2.reshape(1, D),
        Wa1, ba1.reshape(1, D), Wa2, ba2.reshape(1, D),
        Wa3, ba3.reshape(1, 1),
    )
